# baseline (speedup 1.0000x reference)
import functools

import jax
import jax.numpy as jnp
from jax import lax
from jax.experimental import pallas as pl
from jax.experimental.pallas import tpu as pltpu
from jax.experimental.pallas import tpu_sc as plsc

EMBED_D = 128
N_ROWS = 100000
NUM_CORES = 2
NUM_SUBCORES = 16
NUM_WORKERS = NUM_CORES * NUM_SUBCORES
CHUNK = 400
NUM_CHUNKS = N_ROWS // CHUNK
MAX_ITERS = -(-NUM_CHUNKS // NUM_WORKERS)

_mesh = plsc.VectorSubcoreMesh(
    core_axis_name="c", subcore_axis_name="s",
    num_cores=NUM_CORES, num_subcores=NUM_SUBCORES)


@functools.partial(
    pl.kernel,
    mesh=_mesh,
    out_type=jax.ShapeDtypeStruct((N_ROWS, EMBED_D), jnp.float32),
    scratch_types=(
        [pltpu.VMEM((CHUNK,), jnp.int32) for _ in range(MAX_ITERS)]
        + [
            pltpu.VMEM((2, CHUNK, EMBED_D), jnp.float32),
            pltpu.SemaphoreType.DMA,
            pltpu.SemaphoreType.DMA,
            pltpu.SemaphoreType.DMA,
        ]
    ),
)
def _embed_sc(idx_hbm, table_hbm, out_hbm, *scratch):
    idx_v = scratch[:MAX_ITERS]
    rows_v, sem_g, sem_s0, sem_s1 = scratch[MAX_ITERS:]
    wid = lax.axis_index("s") * NUM_CORES + lax.axis_index("c")
    sem_s = (sem_s0, sem_s1)

    def chunk_id(k):
        return wid + k * NUM_WORKERS

    def out_slice(k):
        return out_hbm.at[pl.ds(chunk_id(k) * CHUNK, CHUNK)]

    for k in range(MAX_ITERS):

        @pl.when(chunk_id(k) < NUM_CHUNKS)
        def _():
            pltpu.sync_copy(idx_hbm.at[pl.ds(chunk_id(k) * CHUNK, CHUNK)],
                            idx_v[k])

    for k in range(MAX_ITERS):
        buf = k % 2

        @pl.when(chunk_id(k) < NUM_CHUNKS)
        def _():
            if k >= 2:
                pltpu.make_async_copy(rows_v.at[buf], out_slice(k - 2),
                                      sem_s[buf]).wait()
            pltpu.async_copy(table_hbm.at[idx_v[k]], rows_v.at[buf],
                             sem_g).wait()
            pltpu.async_copy(rows_v.at[buf], out_slice(k), sem_s[buf])

    for k in range(max(MAX_ITERS - 2, 0), MAX_ITERS):
        buf = k % 2

        @pl.when(chunk_id(k) < NUM_CHUNKS)
        def _():
            pltpu.make_async_copy(rows_v.at[buf], out_slice(k),
                                  sem_s[buf]).wait()


def kernel(atom_type_index, embedding_table):
    idx = atom_type_index.astype(jnp.int32)
    return _embed_sc(idx, embedding_table)


# speedup vs baseline: 6.5466x; 6.5466x over previous
import functools

import jax
import jax.numpy as jnp
from jax import lax
from jax.experimental import pallas as pl
from jax.experimental.pallas import tpu as pltpu
from jax.experimental.pallas import tpu_sc as plsc

EMBED_D = 128
TABLE_ROWS = 20
N_ROWS = 100000
NUM_CORES = 2
NUM_SUBCORES = 16
NUM_WORKERS = NUM_CORES * NUM_SUBCORES
CHUNK = 400
NUM_CHUNKS = N_ROWS // CHUNK
MAX_ITERS = -(-NUM_CHUNKS // NUM_WORKERS)

_mesh = plsc.VectorSubcoreMesh(
    core_axis_name="c", subcore_axis_name="s",
    num_cores=NUM_CORES, num_subcores=NUM_SUBCORES)


@functools.partial(
    pl.kernel,
    mesh=_mesh,
    out_type=jax.ShapeDtypeStruct((N_ROWS, EMBED_D), jnp.float32),
    scratch_types=(
        [pltpu.VMEM_SHARED((TABLE_ROWS, EMBED_D), jnp.float32),
         pltpu.VMEM((2, CHUNK, EMBED_D), jnp.float32)]
        + [pltpu.VMEM((CHUNK,), jnp.int32) for _ in range(MAX_ITERS)]
        + [pltpu.SemaphoreType.DMA,
           pltpu.SemaphoreType.DMA,
           pltpu.SemaphoreType.DMA]
    ),
)
def _embed_sc(idx_hbm, table_hbm, out_hbm, *scratch):
    table_v, rows_v = scratch[0], scratch[1]
    idx_v = scratch[2:2 + MAX_ITERS]
    sem_g, sem_s0, sem_s1 = scratch[2 + MAX_ITERS:]
    sem_s = (sem_s0, sem_s1)
    wid = lax.axis_index("s") * NUM_CORES + lax.axis_index("c")

    def chunk_id(k):
        return wid + k * NUM_WORKERS

    def out_slice(k):
        return out_hbm.at[pl.ds(chunk_id(k) * CHUNK, CHUNK)]

    @pl.when(lax.axis_index("s") == 0)
    def _():
        pltpu.sync_copy(table_hbm, table_v)

    plsc.subcore_barrier()

    for k in range(MAX_ITERS):

        @pl.when(chunk_id(k) < NUM_CHUNKS)
        def _():
            pltpu.sync_copy(idx_hbm.at[pl.ds(chunk_id(k) * CHUNK, CHUNK)],
                            idx_v[k])

    for k in range(MAX_ITERS):
        buf = k % 2

        @pl.when(chunk_id(k) < NUM_CHUNKS)
        def _():
            if k >= 2:
                pltpu.make_async_copy(rows_v.at[buf], out_slice(k - 2),
                                      sem_s[buf]).wait()
            pltpu.async_copy(table_v.at[idx_v[k]], rows_v.at[buf],
                             sem_g).wait()
            pltpu.async_copy(rows_v.at[buf], out_slice(k), sem_s[buf])

    for k in range(max(MAX_ITERS - 2, 0), MAX_ITERS):
        buf = k % 2

        @pl.when(chunk_id(k) < NUM_CHUNKS)
        def _():
            pltpu.make_async_copy(rows_v.at[buf], out_slice(k),
                                  sem_s[buf]).wait()


def kernel(atom_type_index, embedding_table):
    idx = atom_type_index.astype(jnp.int32)
    return _embed_sc(idx, embedding_table)


# speedup vs baseline: 7.0789x; 1.0813x over previous
import functools

import jax
import jax.numpy as jnp
from jax import lax
from jax.experimental import pallas as pl
from jax.experimental.pallas import tpu as pltpu
from jax.experimental.pallas import tpu_sc as plsc

EMBED_D = 128
TABLE_ROWS = 20
N_ROWS = 100000
NUM_CORES = 2
NUM_SUBCORES = 16
NUM_WORKERS = NUM_CORES * NUM_SUBCORES
CHUNK = 400
NUM_CHUNKS = N_ROWS // CHUNK
MAX_ITERS = -(-NUM_CHUNKS // NUM_WORKERS)

_mesh = plsc.VectorSubcoreMesh(
    core_axis_name="c", subcore_axis_name="s",
    num_cores=NUM_CORES, num_subcores=NUM_SUBCORES)


@functools.partial(
    pl.kernel,
    mesh=_mesh,
    out_type=jax.ShapeDtypeStruct((N_ROWS, EMBED_D), jnp.float32),
    scratch_types=(
        [pltpu.VMEM_SHARED((TABLE_ROWS, EMBED_D), jnp.float32),
         pltpu.VMEM((2, CHUNK, EMBED_D), jnp.float32)]
        + [pltpu.VMEM((CHUNK,), jnp.int32) for _ in range(MAX_ITERS)]
        + [pltpu.SemaphoreType.DMA,
           pltpu.SemaphoreType.DMA,
           pltpu.SemaphoreType.DMA,
           pltpu.SemaphoreType.DMA]
    ),
)
def _embed_sc(idx_hbm, table_hbm, out_hbm, *scratch):
    table_v, rows_v = scratch[0], scratch[1]
    idx_v = scratch[2:2 + MAX_ITERS]
    sem_g, sem_s0, sem_s1, sem_i = scratch[2 + MAX_ITERS:]
    sem_s = (sem_s0, sem_s1)
    wid = lax.axis_index("s") * NUM_CORES + lax.axis_index("c")

    def chunk_id(k):
        return wid + k * NUM_WORKERS

    def out_slice(k):
        return out_hbm.at[pl.ds(chunk_id(k) * CHUNK, CHUNK)]

    @pl.when(lax.axis_index("s") == 0)
    def _():
        pltpu.sync_copy(table_hbm, table_v)

    plsc.subcore_barrier()

    for k in range(MAX_ITERS):

        @pl.when(chunk_id(k) < NUM_CHUNKS)
        def _():
            pltpu.async_copy(idx_hbm.at[pl.ds(chunk_id(k) * CHUNK, CHUNK)],
                             idx_v[k], sem_i)

    for k in range(MAX_ITERS):

        @pl.when(chunk_id(k) < NUM_CHUNKS)
        def _():
            pltpu.make_async_copy(
                idx_hbm.at[pl.ds(chunk_id(k) * CHUNK, CHUNK)],
                idx_v[k], sem_i).wait()

    for k in range(MAX_ITERS):
        buf = k % 2

        @pl.when(chunk_id(k) < NUM_CHUNKS)
        def _():
            if k >= 2:
                pltpu.make_async_copy(rows_v.at[buf], out_slice(k - 2),
                                      sem_s[buf]).wait()
            pltpu.async_copy(table_v.at[idx_v[k]], rows_v.at[buf],
                             sem_g).wait()
            pltpu.async_copy(rows_v.at[buf], out_slice(k), sem_s[buf])

    for k in range(max(MAX_ITERS - 2, 0), MAX_ITERS):
        buf = k % 2

        @pl.when(chunk_id(k) < NUM_CHUNKS)
        def _():
            pltpu.make_async_copy(rows_v.at[buf], out_slice(k),
                                  sem_s[buf]).wait()


def kernel(atom_type_index, embedding_table):
    idx = atom_type_index.astype(jnp.int32)
    return _embed_sc(idx, embedding_table)
